# baseline (speedup 1.0000x reference)
import functools

import jax
import jax.numpy as jnp
from jax import lax
from jax.experimental import pallas as pl
from jax.experimental.pallas import tpu as pltpu
from jax.experimental.pallas import tpu_sc as plsc

B, N, C_IN, D_CODE, K = 8, 1024, 96, 32, 8192
R = B * N
RT = 4096
KC = 2048
COMMITMENT = 0.25


def _argmin_body(x_ref, wenc_ref, cb_ref, cbsq_ref, enc_out, idx_out):
    x = x_ref[...]
    enc = lax.dot_general(
        x.astype(jnp.bfloat16), wenc_ref[...].astype(jnp.bfloat16),
        (((1,), (0,)), ((), ())), preferred_element_type=jnp.float32)
    enc_out[...] = enc
    enc2_16 = (2.0 * enc).astype(jnp.bfloat16)
    rowsq = jnp.sum(enc * enc, axis=1, keepdims=True)
    lane = lax.broadcasted_iota(jnp.int32, (RT, 128), 1)

    bmin = jnp.full((RT,), jnp.inf, jnp.float32)
    bidx = jnp.zeros((RT,), jnp.int32)
    for w in range(K // KC):
        cb = cb_ref[pl.ds(w * KC, KC), :].astype(jnp.bfloat16)
        dot2 = lax.dot_general(
            enc2_16, cb, (((1,), (1,)), ((), ())),
            preferred_element_type=jnp.float32)
        wval = jnp.full((RT, 128), jnp.inf, jnp.float32)
        widx = jnp.zeros((RT, 128), jnp.int32)
        for g in range(KC // 128):
            c0 = w * KC + g * 128
            dg = ((rowsq - dot2[:, g * 128:(g + 1) * 128])
                  + cbsq_ref[:, c0:c0 + 128])
            upd = dg < wval
            widx = jnp.where(upd, lane + c0, widx)
            wval = jnp.where(upd, dg, wval)
        lmin = jnp.min(wval, axis=1)
        tie = wval == lmin[:, None]
        lidx = jnp.min(jnp.where(tie, widx, jnp.int32(2**31 - 1)), axis=1)
        upd2 = lmin < bmin
        bidx = jnp.where(upd2, lidx, bidx)
        bmin = jnp.where(upd2, lmin, bmin).astype(
            jnp.bfloat16).astype(jnp.float32)
    i = pl.program_id(0)
    idx_out[pl.ds(i * (RT // 128), RT // 128), :] = bidx.reshape(RT // 128, 128)


def _encode_argmin(flat, W_enc, codebook, cbsq):
    return pl.pallas_call(
        _argmin_body,
        grid=(R // RT,),
        in_specs=[
            pl.BlockSpec((RT, C_IN), lambda i: (i, 0)),
            pl.BlockSpec((C_IN, D_CODE), lambda i: (0, 0)),
            pl.BlockSpec((K, D_CODE), lambda i: (0, 0)),
            pl.BlockSpec((1, K), lambda i: (0, 0)),
        ],
        out_specs=[
            pl.BlockSpec((RT, D_CODE), lambda i: (i, 0)),
            pl.BlockSpec((R // 128, 128), lambda i: (0, 0)),
        ],
        out_shape=[
            jax.ShapeDtypeStruct((R, D_CODE), jnp.float32),
            jax.ShapeDtypeStruct((R // 128, 128), jnp.int32),
        ],
    )(flat, W_enc, codebook, cbsq)


_NC, _NS = 2, 16
_NW = _NC * _NS
_BPW = R // _NW
_GCH = 128
_DP = 128


@functools.cache
def _sc_gather_kernel():
    mesh = plsc.VectorSubcoreMesh(core_axis_name="c", subcore_axis_name="s")

    @functools.partial(
        pl.kernel,
        mesh=mesh,
        out_type=jax.ShapeDtypeStruct((R, _DP), jnp.float32),
        scratch_types=[
            pltpu.VMEM((_BPW,), jnp.int32),
            pltpu.VMEM((_BPW, _DP), jnp.float32),
            pltpu.SemaphoreType.DMA,
        ],
    )
    def _sc_gather(table_hbm, idx_hbm, out_hbm, idx_v, rows_v, sem):
        wid = lax.axis_index("s") * _NC + lax.axis_index("c")
        base = wid * _BPW
        pltpu.sync_copy(idx_hbm.at[pl.ds(base, _BPW)], idx_v)
        copies = []
        for j in range(_BPW // _GCH):
            copies.append(pltpu.async_copy(
                table_hbm.at[idx_v.at[pl.ds(j * _GCH, _GCH)]],
                rows_v.at[pl.ds(j * _GCH, _GCH)], sem))
        for c in copies:
            c.wait()
        pltpu.sync_copy(rows_v, out_hbm.at[pl.ds(base, _BPW)])

    return _sc_gather


def _decode_body(x_ref, enc_ref, emb_ref, wdec_ref, recon_out, loss_out,
                 emb_out, acc):
    i = pl.program_id(0)

    @pl.when(i == 0)
    def _():
        acc[0] = 0.0
        acc[1] = 0.0

    emb = emb_ref[:, :D_CODE]
    emb_out[...] = emb
    recon = lax.dot_general(
        emb.astype(jnp.bfloat16), wdec_ref[...].astype(jnp.bfloat16),
        (((1,), (0,)), ((), ())), preferred_element_type=jnp.float32)
    recon_out[...] = recon
    d1 = emb - enc_ref[...]
    d2 = recon - x_ref[...]
    acc[0] += jnp.sum(d1 * d1)
    acc[1] += jnp.sum(d2 * d2)

    @pl.when(i == R // RT - 1)
    def _():
        loss_out[...] = jnp.reshape(
            (1.0 + COMMITMENT) * acc[0] / (R * D_CODE)
            + acc[1] / (R * C_IN), (1, 1))


def _decode_loss(flat, enc, emb, W_dec):
    return pl.pallas_call(
        _decode_body,
        grid=(R // RT,),
        in_specs=[
            pl.BlockSpec((RT, C_IN), lambda i: (i, 0)),
            pl.BlockSpec((RT, D_CODE), lambda i: (i, 0)),
            pl.BlockSpec((RT, _DP), lambda i: (i, 0)),
            pl.BlockSpec((D_CODE, C_IN), lambda i: (0, 0)),
        ],
        out_specs=[
            pl.BlockSpec((RT, C_IN), lambda i: (i, 0)),
            pl.BlockSpec((1, 1), lambda i: (0, 0)),
            pl.BlockSpec((RT, D_CODE), lambda i: (i, 0)),
        ],
        out_shape=[
            jax.ShapeDtypeStruct((R, C_IN), jnp.float32),
            jax.ShapeDtypeStruct((1, 1), jnp.float32),
            jax.ShapeDtypeStruct((R, D_CODE), jnp.float32),
        ],
        scratch_shapes=[pltpu.SMEM((2,), jnp.float32)],
    )(flat, enc, emb, W_dec)


def kernel(inputs, W_enc, codebook, W_dec):
    flat = inputs.reshape(R, C_IN)
    cbsq = jnp.sum(codebook ** 2, axis=1)[None, :]
    cb_pad = jnp.pad(codebook, ((0, 0), (0, _DP - D_CODE)))
    enc, idx = _encode_argmin(flat, W_enc, codebook, cbsq)
    emb_pad = _sc_gather_kernel()(cb_pad, idx.reshape(R))
    recon, loss, emb = _decode_loss(flat, enc, emb_pad, W_dec)
    return (loss.reshape(()),
            recon.reshape(B, N, C_IN),
            emb.reshape(B, N, D_CODE))


# speedup vs baseline: 1.0247x; 1.0247x over previous
import functools

import jax
import jax.numpy as jnp
from jax import lax
from jax.experimental import pallas as pl
from jax.experimental.pallas import tpu as pltpu
from jax.experimental.pallas import tpu_sc as plsc

B, N, C_IN, D_CODE, K = 8, 1024, 96, 32, 8192
R = B * N
RT = 2048
KC = 2048
COMMITMENT = 0.25


def _argmin_body(x_ref, wenc_ref, cb_ref, cbsq_ref, enc_out, idx_out):
    x = x_ref[...]
    enc = lax.dot_general(
        x.astype(jnp.bfloat16), wenc_ref[...].astype(jnp.bfloat16),
        (((1,), (0,)), ((), ())), preferred_element_type=jnp.float32)
    enc_out[...] = enc
    enc2_16 = (2.0 * enc).astype(jnp.bfloat16)
    rowsq = jnp.sum(enc * enc, axis=1, keepdims=True)
    lane = lax.broadcasted_iota(jnp.int32, (RT, 128), 1)

    bmin = jnp.full((RT,), jnp.inf, jnp.float32)
    bidx = jnp.zeros((RT,), jnp.int32)
    for w in range(K // KC):
        cb = cb_ref[pl.ds(w * KC, KC), :].astype(jnp.bfloat16)
        dot2 = lax.dot_general(
            enc2_16, cb, (((1,), (1,)), ((), ())),
            preferred_element_type=jnp.float32)
        wval = jnp.full((RT, 128), jnp.inf, jnp.float32)
        widx = jnp.zeros((RT, 128), jnp.int32)
        for g in range(KC // 128):
            c0 = w * KC + g * 128
            dg = ((rowsq - dot2[:, g * 128:(g + 1) * 128])
                  + cbsq_ref[:, c0:c0 + 128])
            upd = dg < wval
            widx = jnp.where(upd, lane + c0, widx)
            wval = jnp.where(upd, dg, wval)
        lmin = jnp.min(wval, axis=1)
        tie = wval == lmin[:, None]
        lidx = jnp.min(jnp.where(tie, widx, jnp.int32(2**31 - 1)), axis=1)
        upd2 = lmin < bmin
        bidx = jnp.where(upd2, lidx, bidx)
        bmin = jnp.where(upd2, lmin, bmin).astype(
            jnp.bfloat16).astype(jnp.float32)
    i = pl.program_id(0)
    idx_out[pl.ds(i * (RT // 128), RT // 128), :] = bidx.reshape(RT // 128, 128)


def _encode_argmin(flat, W_enc, codebook, cbsq):
    return pl.pallas_call(
        _argmin_body,
        grid=(R // RT,),
        in_specs=[
            pl.BlockSpec((RT, C_IN), lambda i: (i, 0)),
            pl.BlockSpec((C_IN, D_CODE), lambda i: (0, 0)),
            pl.BlockSpec((K, D_CODE), lambda i: (0, 0)),
            pl.BlockSpec((1, K), lambda i: (0, 0)),
        ],
        out_specs=[
            pl.BlockSpec((RT, D_CODE), lambda i: (i, 0)),
            pl.BlockSpec((R // 128, 128), lambda i: (0, 0)),
        ],
        out_shape=[
            jax.ShapeDtypeStruct((R, D_CODE), jnp.float32),
            jax.ShapeDtypeStruct((R // 128, 128), jnp.int32),
        ],
    )(flat, W_enc, codebook, cbsq)


_NC, _NS = 2, 16
_NW = _NC * _NS
_BPW = R // _NW
_GCH = 128
_DP = 32


@functools.cache
def _sc_gather_kernel():
    mesh = plsc.VectorSubcoreMesh(core_axis_name="c", subcore_axis_name="s")

    @functools.partial(
        pl.kernel,
        mesh=mesh,
        out_type=jax.ShapeDtypeStruct((R, _DP), jnp.float32),
        scratch_types=[
            pltpu.VMEM((_BPW,), jnp.int32),
            pltpu.VMEM((_BPW, _DP), jnp.float32),
            pltpu.SemaphoreType.DMA,
        ],
        compiler_params=pltpu.CompilerParams(use_tc_tiling_on_sc=False),
    )
    def _sc_gather(table_hbm, idx_hbm, out_hbm, idx_v, rows_v, sem):
        wid = lax.axis_index("s") * _NC + lax.axis_index("c")
        base = wid * _BPW
        pltpu.sync_copy(idx_hbm.at[pl.ds(base, _BPW)], idx_v)
        copies = []
        for j in range(_BPW // _GCH):
            copies.append(pltpu.async_copy(
                table_hbm.at[idx_v.at[pl.ds(j * _GCH, _GCH)]],
                rows_v.at[pl.ds(j * _GCH, _GCH)], sem))
        for c in copies:
            c.wait()
        pltpu.sync_copy(rows_v, out_hbm.at[pl.ds(base, _BPW)])

    return _sc_gather


def _decode_body(x_ref, enc_ref, emb_ref, wdec_ref, recon_out, loss_out,
                 emb_out, acc):
    i = pl.program_id(0)

    @pl.when(i == 0)
    def _():
        acc[0] = 0.0
        acc[1] = 0.0

    emb = emb_ref[:, :D_CODE]
    emb_out[...] = emb
    recon = lax.dot_general(
        emb.astype(jnp.bfloat16), wdec_ref[...].astype(jnp.bfloat16),
        (((1,), (0,)), ((), ())), preferred_element_type=jnp.float32)
    recon_out[...] = recon
    d1 = emb - enc_ref[...]
    d2 = recon - x_ref[...]
    acc[0] += jnp.sum(d1 * d1)
    acc[1] += jnp.sum(d2 * d2)

    @pl.when(i == R // RT - 1)
    def _():
        loss_out[...] = jnp.reshape(
            (1.0 + COMMITMENT) * acc[0] / (R * D_CODE)
            + acc[1] / (R * C_IN), (1, 1))


def _decode_loss(flat, enc, emb, W_dec):
    return pl.pallas_call(
        _decode_body,
        grid=(R // RT,),
        in_specs=[
            pl.BlockSpec((RT, C_IN), lambda i: (i, 0)),
            pl.BlockSpec((RT, D_CODE), lambda i: (i, 0)),
            pl.BlockSpec((RT, _DP), lambda i: (i, 0)),
            pl.BlockSpec((D_CODE, C_IN), lambda i: (0, 0)),
        ],
        out_specs=[
            pl.BlockSpec((RT, C_IN), lambda i: (i, 0)),
            pl.BlockSpec((1, 1), lambda i: (0, 0)),
            pl.BlockSpec((RT, D_CODE), lambda i: (i, 0)),
        ],
        out_shape=[
            jax.ShapeDtypeStruct((R, C_IN), jnp.float32),
            jax.ShapeDtypeStruct((1, 1), jnp.float32),
            jax.ShapeDtypeStruct((R, D_CODE), jnp.float32),
        ],
        scratch_shapes=[pltpu.SMEM((2,), jnp.float32)],
    )(flat, enc, emb, W_dec)


def kernel(inputs, W_enc, codebook, W_dec):
    flat = inputs.reshape(R, C_IN)
    cbsq = jnp.sum(codebook ** 2, axis=1)[None, :]
    cb_pad = jnp.pad(codebook, ((0, 0), (0, _DP - D_CODE)))
    enc, idx = _encode_argmin(flat, W_enc, codebook, cbsq)
    emb_pad = _sc_gather_kernel()(cb_pad, idx.reshape(R))
    recon, loss, emb = _decode_loss(flat, enc, emb_pad, W_dec)
    return (loss.reshape(()),
            recon.reshape(B, N, C_IN),
            emb.reshape(B, N, D_CODE))
